# Initial kernel scaffold; baseline (speedup 1.0000x reference)
import jax
import jax.numpy as jnp
from jax.experimental import pallas as pl


def kernel(x, edge_index, W1, b1, u1, W2, b2, u2, gamma, beta):
    raise NotImplementedError("write your pallas kernel here")


# speedup vs baseline: 6.7341x; 6.7341x over previous
import functools

import jax
import jax.numpy as jnp
from jax import lax
from jax.experimental import pallas as pl
from jax.experimental.pallas import tpu as pltpu
from jax.experimental.pallas import tpu_sc as plsc

N = 10000
E = 320000
F = 128

NC = 2
NS = 16
NW = NC * NS
EPT = E // NW
K = 80
NCHUNK = EPT // K
RPT = N // NS
ZROWS = 125

BLK = 1000
NB = N // BLK


def _sc_segment_sum(x, src3, dst3):
    mesh = plsc.VectorSubcoreMesh(core_axis_name="c", subcore_axis_name="s")

    @functools.partial(
        pl.kernel,
        out_type=jax.ShapeDtypeStruct((NC * N, F), jnp.float32),
        mesh=mesh,
        scratch_types=[
            pltpu.VMEM((NCHUNK, K), jnp.int32),
            pltpu.VMEM((NCHUNK, K), jnp.int32),
            pltpu.VMEM((K, F), jnp.float32),
            pltpu.VMEM((ZROWS, F), jnp.float32),
            pltpu.VMEM_SHARED((N, F), jnp.float32),
            pltpu.SemaphoreType.DMA,
        ],
        compiler_params=pltpu.CompilerParams(use_tc_tiling_on_sc=False),
    )
    def seg_sum(x_hbm, src_hbm, dst_hbm, out_hbm,
                idx_s, idx_d, rows, zbuf, acc, sem):
        cid = lax.axis_index("c")
        sid = lax.axis_index("s")
        wid = sid * NC + cid

        def zrow(r, carry):
            for c in range(F // 16):
                zbuf[r, pl.ds(c * 16, 16)] = jnp.zeros((16,), jnp.float32)
            return carry
        lax.fori_loop(0, ZROWS, zrow, 0)
        base = sid * RPT
        for b in range(RPT // ZROWS):
            pltpu.sync_copy(zbuf, acc.at[pl.ds(base + b * ZROWS, ZROWS)])
        plsc.subcore_barrier()

        pltpu.sync_copy(src_hbm.at[wid], idx_s)
        pltpu.sync_copy(dst_hbm.at[wid], idx_d)

        def body(j, carry):
            pltpu.async_copy(x_hbm.at[idx_s.at[j]], rows, sem).wait()
            pltpu.sync_copy(rows, acc.at[idx_d.at[j]], add=True)
            return carry
        lax.fori_loop(0, NCHUNK, body, 0)
        plsc.subcore_barrier()

        pltpu.sync_copy(acc.at[pl.ds(base, RPT)],
                        out_hbm.at[pl.ds(cid * N + base, RPT)])

    return seg_sum(x, src3, dst3)


def _spectral_sigma(W_ref, u_row):
    Wm = W_ref[...]
    v = lax.dot_general(u_row, Wm, (((1,), (0,)), ((), ())),
                        preferred_element_type=jnp.float32)
    v = v / (jnp.sqrt(jnp.sum(v * v)) + 1e-12)
    w = lax.dot_general(v, Wm, (((1,), (1,)), ((), ())),
                        preferred_element_type=jnp.float32)
    wn = w / (jnp.sqrt(jnp.sum(w * w)) + 1e-12)
    return jnp.sum(wn * w)


def _tc_stage1(x, p0, p1, W1, b1r, u1r, W2, u2r):
    def body(x_ref, p0_ref, p1_ref, W1_ref, b1_ref, u1_ref, W2_ref, u2_ref,
             y_ref, st_ref, acc_ref, sig_ref):
        i = pl.program_id(0)

        @pl.when(i == 0)
        def _():
            sig_ref[0] = _spectral_sigma(W1_ref, u1_ref[...])
            sig_ref[1] = _spectral_sigma(W2_ref, u2_ref[...])
            acc_ref[...] = jnp.zeros_like(acc_ref)

        sig1 = sig_ref[0]
        h = x_ref[...] + p0_ref[...] + p1_ref[...]
        y = lax.dot_general(h, W1_ref[...], (((1,), (1,)), ((), ())),
                            preferred_element_type=jnp.float32)
        y = jnp.maximum(y + sig1 * b1_ref[...], 0.0)
        y_ref[...] = y
        acc_ref[0:1, :] += jnp.sum(y, axis=0, keepdims=True)
        acc_ref[1:2, :] += jnp.sum(y * y, axis=0, keepdims=True)

        @pl.when(i == NB - 1)
        def _():
            st_ref[0:2, :] = acc_ref[0:2, :]
            st_ref[2:3, :] = jnp.full((1, F), sig_ref[0], jnp.float32)
            st_ref[3:4, :] = jnp.full((1, F), sig_ref[1], jnp.float32)
            st_ref[4:8, :] = jnp.zeros((4, F), jnp.float32)

    return pl.pallas_call(
        body,
        grid=(NB,),
        in_specs=[
            pl.BlockSpec((BLK, F), lambda i: (i, 0)),
            pl.BlockSpec((BLK, F), lambda i: (i, 0)),
            pl.BlockSpec((BLK, F), lambda i: (i, 0)),
            pl.BlockSpec((F, F), lambda i: (0, 0)),
            pl.BlockSpec((1, F), lambda i: (0, 0)),
            pl.BlockSpec((1, F), lambda i: (0, 0)),
            pl.BlockSpec((F, F), lambda i: (0, 0)),
            pl.BlockSpec((1, F), lambda i: (0, 0)),
        ],
        out_specs=[
            pl.BlockSpec((BLK, F), lambda i: (i, 0)),
            pl.BlockSpec((8, F), lambda i: (0, 0)),
        ],
        out_shape=[
            jax.ShapeDtypeStruct((N, F), jnp.float32),
            jax.ShapeDtypeStruct((8, F), jnp.float32),
        ],
        scratch_shapes=[
            pltpu.VMEM((8, F), jnp.float32),
            pltpu.SMEM((2,), jnp.float32),
        ],
        compiler_params=pltpu.CompilerParams(
            dimension_semantics=("arbitrary",)),
    )(x, p0, p1, W1, b1r, u1r, W2, u2r)


def _tc_stage2(y1, stats, W2, b2r, gr, ber):
    def body(y_ref, st_ref, W2_ref, b2_ref, g_ref, be_ref, o_ref):
        mean = st_ref[0:1, :] * (1.0 / N)
        msq = st_ref[1:2, :] * (1.0 / N)
        var = msq - mean * mean
        sig1 = st_ref[2:3, :]
        sig2 = st_ref[3:4, :]
        rstd = lax.rsqrt(var + 1e-5 * sig1 * sig1)
        scale = g_ref[...] * rstd
        shift = be_ref[...] - mean * scale
        t = y_ref[...] * scale + shift
        o = lax.dot_general(t, W2_ref[...], (((1,), (1,)), ((), ())),
                            preferred_element_type=jnp.float32)
        o_ref[...] = o / sig2 + b2_ref[...]

    return pl.pallas_call(
        body,
        grid=(NB,),
        in_specs=[
            pl.BlockSpec((BLK, F), lambda i: (i, 0)),
            pl.BlockSpec((8, F), lambda i: (0, 0)),
            pl.BlockSpec((F, F), lambda i: (0, 0)),
            pl.BlockSpec((1, F), lambda i: (0, 0)),
            pl.BlockSpec((1, F), lambda i: (0, 0)),
            pl.BlockSpec((1, F), lambda i: (0, 0)),
        ],
        out_specs=pl.BlockSpec((BLK, F), lambda i: (i, 0)),
        out_shape=jax.ShapeDtypeStruct((N, F), jnp.float32),
    )(y1, stats, W2, b2r, gr, ber)


def kernel(x, edge_index, W1, b1, u1, W2, b2, u2, gamma, beta):
    src3 = edge_index[0].astype(jnp.int32).reshape(NW, NCHUNK, K)
    dst3 = edge_index[1].astype(jnp.int32).reshape(NW, NCHUNK, K)
    parts = _sc_segment_sum(x, src3, dst3)
    p0 = parts[:N]
    p1 = parts[N:]
    y1, stats = _tc_stage1(x, p0, p1, W1, b1.reshape(1, F),
                           u1.reshape(1, F), W2, u2.reshape(1, F))
    return _tc_stage2(y1, stats, W2, b2.reshape(1, F),
                      gamma.reshape(1, F), beta.reshape(1, F))


# speedup vs baseline: 10.0274x; 1.4890x over previous
import functools

import jax
import jax.numpy as jnp
from jax import lax
from jax.experimental import pallas as pl
from jax.experimental.pallas import tpu as pltpu
from jax.experimental.pallas import tpu_sc as plsc

N = 10000
E = 320000
F = 128

NC = 2
NS = 16
NW = NC * NS
EPT = E // NW
K = 80
NCHUNK = EPT // K
RPT = N // NS
ZROWS = 25

BLK = 1000
NB = N // BLK


def _sc_segment_sum(x, src3, dst3):
    mesh = plsc.VectorSubcoreMesh(core_axis_name="c", subcore_axis_name="s")

    @functools.partial(
        pl.kernel,
        out_type=jax.ShapeDtypeStruct((NC * N, F), jnp.float32),
        mesh=mesh,
        scratch_types=[
            pltpu.VMEM((NCHUNK, K), jnp.int32),
            pltpu.VMEM((NCHUNK, K), jnp.int32),
            pltpu.VMEM((K, F), jnp.float32),
            pltpu.VMEM((K, F), jnp.float32),
            pltpu.VMEM((ZROWS, F), jnp.float32),
            pltpu.VMEM_SHARED((N, F), jnp.float32),
            pltpu.SemaphoreType.DMA,
            pltpu.SemaphoreType.DMA,
        ],
        compiler_params=pltpu.CompilerParams(use_tc_tiling_on_sc=False),
    )
    def seg_sum(x_hbm, src_hbm, dst_hbm, out_hbm,
                idx_s, idx_d, rows0, rows1, zbuf, acc, sem0, sem1):
        cid = lax.axis_index("c")
        sid = lax.axis_index("s")
        wid = sid * NC + cid

        def zrow(r, carry):
            for c in range(F // 16):
                zbuf[r, pl.ds(c * 16, 16)] = jnp.zeros((16,), jnp.float32)
            return carry
        lax.fori_loop(0, ZROWS, zrow, 0)
        base = sid * RPT
        for b in range(RPT // ZROWS):
            pltpu.sync_copy(zbuf, acc.at[pl.ds(base + b * ZROWS, ZROWS)])
        plsc.subcore_barrier()

        pltpu.sync_copy(src_hbm.at[wid], idx_s)
        pltpu.sync_copy(dst_hbm.at[wid], idx_d)

        pltpu.async_copy(x_hbm.at[idx_s.at[0]], rows0, sem0)

        def body(t, carry):
            c0 = 2 * t
            pltpu.async_copy(x_hbm.at[idx_s.at[c0 + 1]], rows1, sem1)
            pltpu.make_async_copy(x_hbm.at[idx_s.at[c0]], rows0, sem0).wait()
            pltpu.sync_copy(rows0, acc.at[idx_d.at[c0]], add=True)
            pltpu.async_copy(x_hbm.at[idx_s.at[c0 + 2]], rows0, sem0)
            pltpu.make_async_copy(x_hbm.at[idx_s.at[c0 + 1]], rows1,
                                  sem1).wait()
            pltpu.sync_copy(rows1, acc.at[idx_d.at[c0 + 1]], add=True)
            return carry
        lax.fori_loop(0, (NCHUNK - 1) // 2, body, 0)
        pltpu.make_async_copy(x_hbm.at[idx_s.at[NCHUNK - 1]], rows0,
                              sem0).wait()
        pltpu.sync_copy(rows0, acc.at[idx_d.at[NCHUNK - 1]], add=True)
        plsc.subcore_barrier()

        pltpu.sync_copy(acc.at[pl.ds(base, RPT)],
                        out_hbm.at[pl.ds(cid * N + base, RPT)])

    return seg_sum(x, src3, dst3)


def _spectral_sigma(W_ref, u_row):
    Wm = W_ref[...]
    v = lax.dot_general(u_row, Wm, (((1,), (0,)), ((), ())),
                        preferred_element_type=jnp.float32)
    v = v / (jnp.sqrt(jnp.sum(v * v)) + 1e-12)
    w = lax.dot_general(v, Wm, (((1,), (1,)), ((), ())),
                        preferred_element_type=jnp.float32)
    wn = w / (jnp.sqrt(jnp.sum(w * w)) + 1e-12)
    return jnp.sum(wn * w)


def _tc_stage1(x, p0, p1, W1, b1r, u1r, W2, u2r):
    def body(x_ref, p0_ref, p1_ref, W1_ref, b1_ref, u1_ref, W2_ref, u2_ref,
             y_ref, st_ref, acc_ref, sig_ref):
        i = pl.program_id(0)

        @pl.when(i == 0)
        def _():
            sig_ref[0] = _spectral_sigma(W1_ref, u1_ref[...])
            sig_ref[1] = _spectral_sigma(W2_ref, u2_ref[...])
            acc_ref[...] = jnp.zeros_like(acc_ref)

        sig1 = sig_ref[0]
        h = x_ref[...] + p0_ref[...] + p1_ref[...]
        y = lax.dot_general(h, W1_ref[...], (((1,), (1,)), ((), ())),
                            preferred_element_type=jnp.float32)
        y = jnp.maximum(y + sig1 * b1_ref[...], 0.0)
        y_ref[...] = y
        acc_ref[0:1, :] += jnp.sum(y, axis=0, keepdims=True)
        acc_ref[1:2, :] += jnp.sum(y * y, axis=0, keepdims=True)

        @pl.when(i == NB - 1)
        def _():
            st_ref[0:2, :] = acc_ref[0:2, :]
            st_ref[2:3, :] = jnp.full((1, F), sig_ref[0], jnp.float32)
            st_ref[3:4, :] = jnp.full((1, F), sig_ref[1], jnp.float32)
            st_ref[4:8, :] = jnp.zeros((4, F), jnp.float32)

    return pl.pallas_call(
        body,
        grid=(NB,),
        in_specs=[
            pl.BlockSpec((BLK, F), lambda i: (i, 0)),
            pl.BlockSpec((BLK, F), lambda i: (i, 0)),
            pl.BlockSpec((BLK, F), lambda i: (i, 0)),
            pl.BlockSpec((F, F), lambda i: (0, 0)),
            pl.BlockSpec((1, F), lambda i: (0, 0)),
            pl.BlockSpec((1, F), lambda i: (0, 0)),
            pl.BlockSpec((F, F), lambda i: (0, 0)),
            pl.BlockSpec((1, F), lambda i: (0, 0)),
        ],
        out_specs=[
            pl.BlockSpec((BLK, F), lambda i: (i, 0)),
            pl.BlockSpec((8, F), lambda i: (0, 0)),
        ],
        out_shape=[
            jax.ShapeDtypeStruct((N, F), jnp.float32),
            jax.ShapeDtypeStruct((8, F), jnp.float32),
        ],
        scratch_shapes=[
            pltpu.VMEM((8, F), jnp.float32),
            pltpu.SMEM((2,), jnp.float32),
        ],
        compiler_params=pltpu.CompilerParams(
            dimension_semantics=("arbitrary",)),
    )(x, p0, p1, W1, b1r, u1r, W2, u2r)


def _tc_stage2(y1, stats, W2, b2r, gr, ber):
    def body(y_ref, st_ref, W2_ref, b2_ref, g_ref, be_ref, o_ref):
        mean = st_ref[0:1, :] * (1.0 / N)
        msq = st_ref[1:2, :] * (1.0 / N)
        var = msq - mean * mean
        sig1 = st_ref[2:3, :]
        sig2 = st_ref[3:4, :]
        rstd = lax.rsqrt(var + 1e-5 * sig1 * sig1)
        scale = g_ref[...] * rstd
        shift = be_ref[...] - mean * scale
        t = y_ref[...] * scale + shift
        o = lax.dot_general(t, W2_ref[...], (((1,), (1,)), ((), ())),
                            preferred_element_type=jnp.float32)
        o_ref[...] = o / sig2 + b2_ref[...]

    return pl.pallas_call(
        body,
        grid=(NB,),
        in_specs=[
            pl.BlockSpec((BLK, F), lambda i: (i, 0)),
            pl.BlockSpec((8, F), lambda i: (0, 0)),
            pl.BlockSpec((F, F), lambda i: (0, 0)),
            pl.BlockSpec((1, F), lambda i: (0, 0)),
            pl.BlockSpec((1, F), lambda i: (0, 0)),
            pl.BlockSpec((1, F), lambda i: (0, 0)),
        ],
        out_specs=pl.BlockSpec((BLK, F), lambda i: (i, 0)),
        out_shape=jax.ShapeDtypeStruct((N, F), jnp.float32),
    )(y1, stats, W2, b2r, gr, ber)


def kernel(x, edge_index, W1, b1, u1, W2, b2, u2, gamma, beta):
    src3 = edge_index[0].astype(jnp.int32).reshape(NW, NCHUNK, K)
    dst3 = edge_index[1].astype(jnp.int32).reshape(NW, NCHUNK, K)
    parts = _sc_segment_sum(x, src3, dst3)
    p0 = parts[:N]
    p1 = parts[N:]
    y1, stats = _tc_stage1(x, p0, p1, W1, b1.reshape(1, F),
                           u1.reshape(1, F), W2, u2.reshape(1, F))
    return _tc_stage2(y1, stats, W2, b2.reshape(1, F),
                      gamma.reshape(1, F), beta.reshape(1, F))


# speedup vs baseline: 11.3523x; 1.1321x over previous
import functools

import jax
import jax.numpy as jnp
from jax import lax
from jax.experimental import pallas as pl
from jax.experimental.pallas import tpu as pltpu
from jax.experimental.pallas import tpu_sc as plsc

N = 10000
E = 320000
F = 128

NC = 2
NS = 16
NW = NC * NS
EPT = E // NW
K = 80
NCHUNK = EPT // K
RPT = N // NS
ZROWS = 25

BLK = 1000
NB = N // BLK


def _sc_segment_sum(x, edge4):
    mesh = plsc.VectorSubcoreMesh(core_axis_name="c", subcore_axis_name="s")

    @functools.partial(
        pl.kernel,
        out_type=jax.ShapeDtypeStruct((NC * N, F), jnp.float32),
        mesh=mesh,
        scratch_types=[
            pltpu.VMEM((NCHUNK, K), jnp.int32),
            pltpu.VMEM((NCHUNK, K), jnp.int32),
            pltpu.VMEM((K, F), jnp.float32),
            pltpu.VMEM((K, F), jnp.float32),
            pltpu.VMEM((ZROWS, F), jnp.float32),
            pltpu.VMEM_SHARED((N, F), jnp.float32),
            pltpu.SemaphoreType.DMA,
            pltpu.SemaphoreType.DMA,
        ],
        compiler_params=pltpu.CompilerParams(use_tc_tiling_on_sc=False),
    )
    def seg_sum(x_hbm, edge_hbm, out_hbm,
                idx_s, idx_d, rows0, rows1, zbuf, acc, sem0, sem1):
        cid = lax.axis_index("c")
        sid = lax.axis_index("s")
        wid = sid * NC + cid

        def zrow(r, carry):
            for c in range(F // 16):
                zbuf[r, pl.ds(c * 16, 16)] = jnp.zeros((16,), jnp.float32)
            return carry
        lax.fori_loop(0, ZROWS, zrow, 0)
        base = sid * RPT
        for b in range(RPT // ZROWS):
            pltpu.sync_copy(zbuf, acc.at[pl.ds(base + b * ZROWS, ZROWS)])
        plsc.subcore_barrier()

        pltpu.sync_copy(edge_hbm.at[0, wid], idx_s)
        pltpu.sync_copy(edge_hbm.at[1, wid], idx_d)

        pltpu.async_copy(x_hbm.at[idx_s.at[0]], rows0, sem0)

        def body(t, carry):
            c0 = 2 * t
            pltpu.async_copy(x_hbm.at[idx_s.at[c0 + 1]], rows1, sem1)
            pltpu.make_async_copy(x_hbm.at[idx_s.at[c0]], rows0, sem0).wait()
            pltpu.sync_copy(rows0, acc.at[idx_d.at[c0]], add=True)
            pltpu.async_copy(x_hbm.at[idx_s.at[c0 + 2]], rows0, sem0)
            pltpu.make_async_copy(x_hbm.at[idx_s.at[c0 + 1]], rows1,
                                  sem1).wait()
            pltpu.sync_copy(rows1, acc.at[idx_d.at[c0 + 1]], add=True)
            return carry
        lax.fori_loop(0, (NCHUNK - 1) // 2, body, 0)
        pltpu.make_async_copy(x_hbm.at[idx_s.at[NCHUNK - 1]], rows0,
                              sem0).wait()
        pltpu.sync_copy(rows0, acc.at[idx_d.at[NCHUNK - 1]], add=True)
        plsc.subcore_barrier()

        pltpu.sync_copy(acc.at[pl.ds(base, RPT)],
                        out_hbm.at[pl.ds(cid * N + base, RPT)])

    return seg_sum(x, edge4)


def _spectral_sigma(W_ref, u_row):
    Wm = W_ref[...]
    v = lax.dot_general(u_row, Wm, (((1,), (0,)), ((), ())),
                        preferred_element_type=jnp.float32)
    v = v / (jnp.sqrt(jnp.sum(v * v)) + 1e-12)
    w = lax.dot_general(v, Wm, (((1,), (1,)), ((), ())),
                        preferred_element_type=jnp.float32)
    wn = w / (jnp.sqrt(jnp.sum(w * w)) + 1e-12)
    return jnp.sum(wn * w)


def _tc_mlp(x, parts, W1, b1r, u1r, W2, b2r, u2r, gr, ber):
    def body(x_ref, p0_ref, p1_ref, W1_ref, b1_ref, u1_ref, W2_ref, b2_ref,
             u2_ref, g_ref, be_ref, o_ref, ybuf, acc_ref, sig_ref):
        p = pl.program_id(0)
        i = pl.program_id(1)

        @pl.when(jnp.logical_and(p == 0, i == 0))
        def _():
            sig_ref[0] = _spectral_sigma(W1_ref, u1_ref[...])
            sig_ref[1] = _spectral_sigma(W2_ref, u2_ref[...])
            acc_ref[...] = jnp.zeros_like(acc_ref)

        @pl.when(p == 0)
        def _():
            sig1 = sig_ref[0]
            h = x_ref[...] + p0_ref[...] + p1_ref[...]
            y = lax.dot_general(h, W1_ref[...], (((1,), (1,)), ((), ())),
                                preferred_element_type=jnp.float32)
            y = jnp.maximum(y + sig1 * b1_ref[...], 0.0)
            ybuf[pl.ds(i * BLK, BLK), :] = y
            acc_ref[0:1, :] += jnp.sum(y, axis=0, keepdims=True)
            acc_ref[1:2, :] += jnp.sum(y * y, axis=0, keepdims=True)

        @pl.when(p == 1)
        def _():
            sig1 = sig_ref[0]
            sig2 = sig_ref[1]
            mean = acc_ref[0:1, :] * (1.0 / N)
            msq = acc_ref[1:2, :] * (1.0 / N)
            var = msq - mean * mean
            rstd = lax.rsqrt(var + 1e-5 * sig1 * sig1)
            scale = g_ref[...] * rstd
            shift = be_ref[...] - mean * scale
            t = ybuf[pl.ds(i * BLK, BLK), :] * scale + shift
            o = lax.dot_general(t, W2_ref[...], (((1,), (1,)), ((), ())),
                                preferred_element_type=jnp.float32)
            o_ref[...] = o * (1.0 / sig2) + b2_ref[...]

    cst = lambda p, i: (0, 0)
    ph0 = lambda p, i: ((1 - p) * i, 0)
    return pl.pallas_call(
        body,
        grid=(2, NB),
        in_specs=[
            pl.BlockSpec((BLK, F), ph0),
            pl.BlockSpec((BLK, F), ph0),
            pl.BlockSpec((BLK, F), lambda p, i: ((1 - p) * i + NB, 0)),
            pl.BlockSpec((F, F), cst),
            pl.BlockSpec((1, F), cst),
            pl.BlockSpec((1, F), cst),
            pl.BlockSpec((F, F), cst),
            pl.BlockSpec((1, F), cst),
            pl.BlockSpec((1, F), cst),
            pl.BlockSpec((1, F), cst),
            pl.BlockSpec((1, F), cst),
        ],
        out_specs=pl.BlockSpec((BLK, F), lambda p, i: (i, 0)),
        out_shape=jax.ShapeDtypeStruct((N, F), jnp.float32),
        scratch_shapes=[
            pltpu.VMEM((N, F), jnp.float32),
            pltpu.VMEM((8, F), jnp.float32),
            pltpu.SMEM((2,), jnp.float32),
        ],
        compiler_params=pltpu.CompilerParams(
            dimension_semantics=("arbitrary", "arbitrary")),
    )(x, parts, parts, W1, b1r, u1r, W2, b2r, u2r, gr, ber)


def kernel(x, edge_index, W1, b1, u1, W2, b2, u2, gamma, beta):
    edge4 = edge_index.astype(jnp.int32).reshape(2, NW, NCHUNK, K)
    parts = _sc_segment_sum(x, edge4)
    return _tc_mlp(x, parts, W1, b1.reshape(1, F), u1.reshape(1, F),
                   W2, b2.reshape(1, F), u2.reshape(1, F),
                   gamma.reshape(1, F), beta.reshape(1, F))
